# baseline (speedup 1.0000x reference)
import functools

import jax
import jax.numpy as jnp
from jax import lax
from jax.experimental import pallas as pl
from jax.experimental.pallas import tpu as pltpu

_VMEM_LIMIT_BYTES = 48 * 1024 * 1024


def _conv_kernel(x_ref, s_ref, w_ref, b_ref, o_ref, *, hout, wout, m):
    cin, h, w = x_ref.shape[1:]
    xb = x_ref[0].reshape(cin * h, w).astype(jnp.bfloat16)
    p = jnp.dot(xb, s_ref[...], preferred_element_type=jnp.float32)
    pb = p.astype(jnp.bfloat16)
    pi = pltpu.bitcast(pb, jnp.int32)
    even_rows = lax.bitcast_convert_type(
        pi.astype(jnp.int16), jnp.bfloat16)
    odd_rows = lax.bitcast_convert_type(
        lax.shift_right_logical(pi, jnp.int32(16)).astype(jnp.int16),
        jnp.bfloat16)
    er3 = even_rows.reshape(cin, hout, w)
    or3 = odd_rows.reshape(cin, hout, w)

    ee = er3[:, :, :wout].reshape(cin, m)
    eo = er3[:, :, wout:].reshape(cin, m)
    oe = or3[:, :, :wout].reshape(cin, m)
    oo = or3[:, :, wout:].reshape(cin, m)

    lane = lax.broadcasted_iota(jnp.int32, (1, m), 1)
    col0 = (lane % wout) == 0

    def shift_right(a, s):
        return jnp.concatenate(
            [jnp.zeros((cin, s), a.dtype), a[:, :m - s]], axis=-1)

    def mask_col0(a):
        return jnp.where(col0, jnp.zeros((), a.dtype), a)

    taps = (
        mask_col0(shift_right(oo, wout + 1)),
        shift_right(oe, wout),
        shift_right(oo, wout),
        mask_col0(shift_right(eo, 1)),
        ee,
        eo,
        mask_col0(shift_right(oo, 1)),
        oe,
        oo,
    )

    cout = b_ref.shape[0]
    acc = jnp.dot(w_ref[0:cout, :], taps[0],
                  preferred_element_type=jnp.float32)
    for t in range(1, 9):
        acc += jnp.dot(w_ref[t * cout:(t + 1) * cout, :], taps[t],
                       preferred_element_type=jnp.float32)
    o_ref[0] = (acc + b_ref[...]).reshape(o_ref.shape[1:])


def kernel(x_nchw, w_oihw, bias):
    n, cin, h, w = x_nchw.shape
    cout = w_oihw.shape[0]
    hout, wout = h // 2, w // 2
    m = hout * wout

    v = jnp.arange(w)
    src = jnp.where(v < wout, 2 * v, 2 * (v - wout) + 1)
    smat = (jnp.arange(w)[:, None] == src[None, :]).astype(jnp.bfloat16)

    w2 = jnp.transpose(w_oihw, (2, 3, 0, 1)).reshape(9 * cout, cin)
    w2 = w2.astype(jnp.bfloat16)
    b2 = bias.astype(jnp.float32).reshape(cout, 1)

    cost = pl.CostEstimate(
        flops=2 * n * m * (9 * cin * cout + 2 * cin * w),
        transcendentals=0,
        bytes_accessed=x_nchw.size * 4 + w2.size * 2 + n * cout * m * 4,
    )

    out = pl.pallas_call(
        functools.partial(_conv_kernel, hout=hout, wout=wout, m=m),
        out_shape=jax.ShapeDtypeStruct((n, cout, hout, wout), jnp.float32),
        grid=(n,),
        in_specs=[
            pl.BlockSpec((1, cin, h, w), lambda i: (i, 0, 0, 0)),
            pl.BlockSpec((w, w), lambda i: (0, 0)),
            pl.BlockSpec((9 * cout, cin), lambda i: (0, 0)),
            pl.BlockSpec((cout, 1), lambda i: (0, 0)),
        ],
        out_specs=pl.BlockSpec((1, cout, hout, wout), lambda i: (i, 0, 0, 0)),
        compiler_params=pltpu.CompilerParams(
            dimension_semantics=("parallel",),
            vmem_limit_bytes=_VMEM_LIMIT_BYTES),
        cost_estimate=cost,
    )(x_nchw, smat, w2, b2)

    return out.astype(x_nchw.dtype)


# speedup vs baseline: 1.2631x; 1.2631x over previous
import functools

import jax
import jax.numpy as jnp
from jax import lax
from jax.experimental import pallas as pl
from jax.experimental.pallas import tpu as pltpu

_VMEM_LIMIT_BYTES = 48 * 1024 * 1024


def _conv_kernel(x_ref, s_ref, w_ref, b_ref, o_ref, *, hout, wout, m):
    nb = x_ref.shape[0]
    cin, h, w = x_ref.shape[1:]
    for b in range(nb):
        _conv_one(x_ref, s_ref, w_ref, b_ref, o_ref, b,
                  hout=hout, wout=wout, m=m)


def _conv_one(x_ref, s_ref, w_ref, b_ref, o_ref, b, *, hout, wout, m):
    cin, h, w = x_ref.shape[1:]
    xb = x_ref[b].reshape(cin * h, w).astype(jnp.bfloat16)
    p = jnp.dot(xb, s_ref[...], preferred_element_type=jnp.float32)
    pb = p.astype(jnp.bfloat16)
    pi = pltpu.bitcast(pb, jnp.int32)
    even_rows = lax.bitcast_convert_type(
        pi.astype(jnp.int16), jnp.bfloat16)
    odd_rows = lax.bitcast_convert_type(
        lax.shift_right_logical(pi, jnp.int32(16)).astype(jnp.int16),
        jnp.bfloat16)
    er3 = even_rows.reshape(cin, hout, w)
    or3 = odd_rows.reshape(cin, hout, w)

    ee = er3[:, :, :wout].reshape(cin, m)
    eo = er3[:, :, wout:].reshape(cin, m)
    oe = or3[:, :, :wout].reshape(cin, m)
    oo = or3[:, :, wout:].reshape(cin, m)

    lane = lax.broadcasted_iota(jnp.int32, (1, m), 1)
    col0 = (lane % wout) == 0

    def shift_right(a, s):
        return jnp.concatenate(
            [jnp.zeros((cin, s), a.dtype), a[:, :m - s]], axis=-1)

    def mask_col0(a):
        return jnp.where(col0, jnp.zeros((), a.dtype), a)

    taps = (
        mask_col0(shift_right(oo, wout + 1)),
        shift_right(oe, wout),
        shift_right(oo, wout),
        mask_col0(shift_right(eo, 1)),
        ee,
        eo,
        mask_col0(shift_right(oo, 1)),
        oe,
        oo,
    )

    cout = b_ref.shape[0]
    acc = jnp.dot(w_ref[0:cout, :], taps[0],
                  preferred_element_type=jnp.float32)
    for t in range(1, 9):
        acc += jnp.dot(w_ref[t * cout:(t + 1) * cout, :], taps[t],
                       preferred_element_type=jnp.float32)
    o_ref[b] = acc + b_ref[...]


def kernel(x_nchw, w_oihw, bias):
    n, cin, h, w = x_nchw.shape
    cout = w_oihw.shape[0]
    hout, wout = h // 2, w // 2
    m = hout * wout

    v = jnp.arange(w)
    src = jnp.where(v < wout, 2 * v, 2 * (v - wout) + 1)
    smat = (jnp.arange(w)[:, None] == src[None, :]).astype(jnp.bfloat16)

    w2 = jnp.transpose(w_oihw, (2, 3, 0, 1)).reshape(9 * cout, cin)
    w2 = w2.astype(jnp.bfloat16)
    b2 = bias.astype(jnp.float32).reshape(cout, 1)

    cost = pl.CostEstimate(
        flops=2 * n * m * (9 * cin * cout + 2 * cin * w),
        transcendentals=0,
        bytes_accessed=x_nchw.size * 4 + w2.size * 2 + n * cout * m * 4,
    )

    out = pl.pallas_call(
        functools.partial(_conv_kernel, hout=hout, wout=wout, m=m),
        out_shape=jax.ShapeDtypeStruct((n, cout, m), jnp.float32),
        grid=(n // 2,),
        in_specs=[
            pl.BlockSpec((2, cin, h, w), lambda i: (i, 0, 0, 0)),
            pl.BlockSpec((w, w), lambda i: (0, 0)),
            pl.BlockSpec((9 * cout, cin), lambda i: (0, 0)),
            pl.BlockSpec((cout, 1), lambda i: (0, 0)),
        ],
        out_specs=pl.BlockSpec((2, cout, m), lambda i: (i, 0, 0)),
        compiler_params=pltpu.CompilerParams(
            dimension_semantics=("parallel",),
            vmem_limit_bytes=_VMEM_LIMIT_BYTES),
        cost_estimate=cost,
    )(x_nchw, smat, w2, b2)

    return out.reshape(n, cout, hout, wout).astype(x_nchw.dtype)


# speedup vs baseline: 1.2898x; 1.0211x over previous
import functools

import jax
import jax.numpy as jnp
from jax import lax
from jax.experimental import pallas as pl
from jax.experimental.pallas import tpu as pltpu

_VMEM_LIMIT_BYTES = 48 * 1024 * 1024


def _conv_kernel(x_ref, s_ref, w_ref, b_ref, o_ref, *, hout, wout, m):
    nb = x_ref.shape[0]
    cin, h, w = x_ref.shape[1:]
    for b in range(nb):
        _conv_one(x_ref, s_ref, w_ref, b_ref, o_ref, b,
                  hout=hout, wout=wout, m=m)


def _conv_one(x_ref, s_ref, w_ref, b_ref, o_ref, b, *, hout, wout, m):
    cin, h, w = x_ref.shape[1:]
    xb = x_ref[b].reshape(cin * h, w).astype(jnp.bfloat16)
    p = jnp.dot(xb, s_ref[...], preferred_element_type=jnp.float32)
    pb = p.astype(jnp.bfloat16)
    pi = pltpu.bitcast(pb, jnp.int32)
    even_rows = lax.bitcast_convert_type(
        pi.astype(jnp.int16), jnp.bfloat16)
    odd_rows = lax.bitcast_convert_type(
        lax.shift_right_logical(pi, jnp.int32(16)).astype(jnp.int16),
        jnp.bfloat16)
    er3 = even_rows.reshape(cin, hout, w)
    or3 = odd_rows.reshape(cin, hout, w)

    ee = er3[:, :, :wout].reshape(cin, m)
    eo = er3[:, :, wout:].reshape(cin, m)
    oe = or3[:, :, :wout].reshape(cin, m)
    oo = or3[:, :, wout:].reshape(cin, m)

    lane = lax.broadcasted_iota(jnp.int32, (1, m), 1)
    col0 = (lane % wout) == 0

    def shift_right(a, s):
        return jnp.concatenate(
            [jnp.zeros((cin, s), a.dtype), a[:, :m - s]], axis=-1)

    def mask_col0(a):
        return jnp.where(col0, jnp.zeros((), a.dtype), a)

    taps = (
        mask_col0(shift_right(oo, wout + 1)),
        shift_right(oe, wout),
        shift_right(oo, wout),
        mask_col0(shift_right(eo, 1)),
        ee,
        eo,
        mask_col0(shift_right(oo, 1)),
        oe,
        oo,
    )

    cout = b_ref.shape[0]
    acc = jnp.dot(w_ref[0:cout, :], taps[0],
                  preferred_element_type=jnp.float32)
    for t in range(1, 9):
        acc += jnp.dot(w_ref[t * cout:(t + 1) * cout, :], taps[t],
                       preferred_element_type=jnp.float32)
    o_ref[b] = acc + b_ref[...]


def kernel(x_nchw, w_oihw, bias):
    n, cin, h, w = x_nchw.shape
    cout = w_oihw.shape[0]
    hout, wout = h // 2, w // 2
    m = hout * wout

    v = jnp.arange(w)
    src = jnp.where(v < wout, 2 * v, 2 * (v - wout) + 1)
    smat = (jnp.arange(w)[:, None] == src[None, :]).astype(jnp.bfloat16)

    w2 = jnp.transpose(w_oihw, (2, 3, 0, 1)).reshape(9 * cout, cin)
    w2 = w2.astype(jnp.bfloat16)
    b2 = bias.astype(jnp.float32).reshape(cout, 1)

    cost = pl.CostEstimate(
        flops=2 * n * m * (9 * cin * cout + 2 * cin * w),
        transcendentals=0,
        bytes_accessed=x_nchw.size * 4 + w2.size * 2 + n * cout * m * 4,
    )

    out = pl.pallas_call(
        functools.partial(_conv_kernel, hout=hout, wout=wout, m=m),
        out_shape=jax.ShapeDtypeStruct((n, cout, m), jnp.float32),
        grid=(n // 4,),
        in_specs=[
            pl.BlockSpec((4, cin, h, w), lambda i: (i, 0, 0, 0)),
            pl.BlockSpec((w, w), lambda i: (0, 0)),
            pl.BlockSpec((9 * cout, cin), lambda i: (0, 0)),
            pl.BlockSpec((cout, 1), lambda i: (0, 0)),
        ],
        out_specs=pl.BlockSpec((4, cout, m), lambda i: (i, 0, 0)),
        compiler_params=pltpu.CompilerParams(
            dimension_semantics=("parallel",),
            vmem_limit_bytes=_VMEM_LIMIT_BYTES),
        cost_estimate=cost,
    )(x_nchw, smat, w2, b2)

    return out.reshape(n, cout, hout, wout).astype(x_nchw.dtype)
